# baseline (speedup 1.0000x reference)
import functools

import jax
import jax.numpy as jnp
from jax import lax
from jax.experimental import pallas as pl
from jax.experimental.pallas import tpu as pltpu
from jax.experimental.pallas import tpu_sc as plsc

B, C, N = 8, 64, 512
NUM_TYPES, D_EMB = 8, 128

_info = plsc.get_sparse_core_info()
NC, NS, L = _info.num_cores, _info.num_subcores, _info.num_lanes
NW = NC * NS

TOTAL = B * C * N * D_EMB
PER_W = TOTAL // NW
ROWS_BUF = 512
HEAD_ROWS = 128
BUF_ELEMS = ROWS_BUF * D_EMB
N_WRITE = PER_W // BUF_ELEMS


@functools.partial(
    pl.kernel,
    mesh=plsc.VectorSubcoreMesh(core_axis_name="c", subcore_axis_name="s"),
    compiler_params=pltpu.CompilerParams(needs_layout_passes=False),
    out_type=jax.ShapeDtypeStruct((TOTAL,), jnp.float32),
    scratch_types=[
        pltpu.VMEM((L,), jnp.int32),
        pltpu.VMEM((NUM_TYPES, D_EMB), jnp.float32),
        pltpu.VMEM((BUF_ELEMS,), jnp.float32),
        pltpu.SemaphoreType.DMA,
    ],
)
def _emb_broadcast(emb_hbm, idx_hbm, out_hbm, idx_v, emb_v, rows_v, sem_w):
    wid = lax.axis_index("s") * NC + lax.axis_index("c")
    my_b = wid // (NW // B)

    pltpu.sync_copy(idx_hbm, idx_v)
    pltpu.sync_copy(emb_hbm, emb_v)

    row_vec = plsc.load_gather(idx_v, [jnp.full((L,), my_b, jnp.int32)])
    lanes = lax.iota(jnp.int32, L)
    chunks = [
        plsc.load_gather(emb_v, [row_vec, j * L + lanes])
        for j in range(D_EMB // L)
    ]

    base = wid * PER_W
    head_elems = HEAD_ROWS * D_EMB
    for i in range(HEAD_ROWS):
        for j, ch in enumerate(chunks):
            rows_v[pl.ds(i * D_EMB + j * L, L)] = ch
    writes = [
        pltpu.async_copy(rows_v.at[pl.ds(0, head_elems)],
                         out_hbm.at[pl.ds(base, head_elems)], sem_w)
    ]
    for i in range(HEAD_ROWS, ROWS_BUF):
        for j, ch in enumerate(chunks):
            rows_v[pl.ds(i * D_EMB + j * L, L)] = ch
    writes.append(
        pltpu.async_copy(rows_v.at[pl.ds(head_elems, BUF_ELEMS - head_elems)],
                         out_hbm.at[pl.ds(base + head_elems,
                                          BUF_ELEMS - head_elems)], sem_w))

    writes += [
        pltpu.async_copy(rows_v, out_hbm.at[pl.ds(base + i * BUF_ELEMS, BUF_ELEMS)],
                         sem_w)
        for i in range(1, N_WRITE)
    ]
    for cp in writes:
        cp.wait()


def kernel(x, emb_table, ch_indices):
    del x
    idx16 = jnp.pad(ch_indices.astype(jnp.int32), (0, L - B))
    out = _emb_broadcast(emb_table.astype(jnp.float32), idx16)
    return out.reshape(B, C, N, D_EMB)


# speedup vs baseline: 1.2106x; 1.2106x over previous
import functools

import jax
import jax.numpy as jnp
from jax import lax
from jax.experimental import pallas as pl
from jax.experimental.pallas import tpu as pltpu
from jax.experimental.pallas import tpu_sc as plsc

B, C, N = 8, 64, 512
NUM_TYPES, D_EMB = 8, 128

_info = plsc.get_sparse_core_info()
NC, NS, L = _info.num_cores, _info.num_subcores, _info.num_lanes
NW = NC * NS

TOTAL = B * C * N * D_EMB
PER_W = TOTAL // NW
ROWS_BUF = 512
HEAD_ROWS = 128
BUF_ELEMS = ROWS_BUF * D_EMB
N_WRITE = PER_W // BUF_ELEMS


@functools.partial(
    pl.kernel,
    mesh=plsc.VectorSubcoreMesh(core_axis_name="c", subcore_axis_name="s"),
    compiler_params=pltpu.CompilerParams(needs_layout_passes=False),
    out_type=jax.ShapeDtypeStruct((TOTAL,), jnp.float32),
    scratch_types=[
        pltpu.VMEM((L,), jnp.int32),
        pltpu.VMEM((NUM_TYPES, D_EMB), jnp.float32),
        pltpu.VMEM((BUF_ELEMS,), jnp.float32),
        pltpu.SemaphoreType.DMA,
    ],
)
def _emb_broadcast(emb_hbm, idx_hbm, out_hbm, idx_v, emb_v, rows_v, sem_w):
    wid = lax.axis_index("s") * NC + lax.axis_index("c")
    my_b = wid // (NW // B)

    pltpu.sync_copy(idx_hbm, idx_v)
    pltpu.sync_copy(emb_hbm, emb_v)

    row_vec = plsc.load_gather(idx_v, [jnp.full((L,), my_b, jnp.int32)])
    lanes = lax.iota(jnp.int32, L)
    chunks = [
        plsc.load_gather(emb_v, [row_vec, j * L + lanes])
        for j in range(D_EMB // L)
    ]

    base = wid * PER_W
    head_elems = HEAD_ROWS * D_EMB

    def fill(i, _):
        row = i * D_EMB
        for j, ch in enumerate(chunks):
            rows_v[pl.ds(row + j * L, L)] = ch
        return 0

    lax.fori_loop(0, HEAD_ROWS, fill, 0)
    writes = [
        pltpu.async_copy(rows_v.at[pl.ds(0, head_elems)],
                         out_hbm.at[pl.ds(base, head_elems)], sem_w)
    ]
    lax.fori_loop(HEAD_ROWS, ROWS_BUF, fill, 0)
    writes.append(
        pltpu.async_copy(rows_v.at[pl.ds(head_elems, BUF_ELEMS - head_elems)],
                         out_hbm.at[pl.ds(base + head_elems,
                                          BUF_ELEMS - head_elems)], sem_w))

    writes += [
        pltpu.async_copy(rows_v, out_hbm.at[pl.ds(base + i * BUF_ELEMS, BUF_ELEMS)],
                         sem_w)
        for i in range(1, N_WRITE)
    ]
    for cp in writes:
        cp.wait()


def kernel(x, emb_table, ch_indices):
    del x
    idx16 = jnp.pad(ch_indices.astype(jnp.int32), (0, L - B))
    out = _emb_broadcast(emb_table.astype(jnp.float32), idx16)
    return out.reshape(B, C, N, D_EMB)
